# baseline (speedup 1.0000x reference)
import functools

import jax
import jax.numpy as jnp
from jax import lax
from jax.experimental import pallas as pl
from jax.experimental.pallas import tpu as pltpu

_FEAT_C = 64
_OUT_C = 32
_POOL = 64
_GB = 16
_NS = 4


def _body(C, Hf, Wf, x0_ref, x1_ref, x2_ref, x3_ref, ph_ref, m_ref, bp_ref,
          g_ref, bc_ref, o_ref):
    f32 = jnp.float32
    bf16 = jnp.bfloat16
    Hc = x0_ref.shape[-1]
    part = _GB // _NS
    CWf = C * Wf

    hi = lax.broadcasted_iota(jnp.int32, (Hf, Hf), 0)
    ho = lax.broadcasted_iota(jnp.int32, (Hf, Hf), 1)
    s_m1 = (hi == ho - 1).astype(bf16)
    s_p1 = (hi == ho + 1).astype(bf16)

    for hb, x_ref in enumerate((x0_ref, x1_ref, x2_ref, x3_ref)):
        x = x_ref[...]
        y = x.reshape(part * C * Wf, _POOL, Hc).sum(axis=1)

        xp = jnp.dot(y.astype(bf16), ph_ref[...],
                     preferred_element_type=f32)

        for g in range(part):
            xp_g = xp[g * CWf:(g + 1) * CWf, :].astype(bf16)
            f_pre = jnp.dot(m_ref[...], xp_g,
                            preferred_element_type=f32)
            feat = jnp.maximum(f_pre + bp_ref[...], 0.0).astype(bf16)

            f_m1 = jnp.dot(feat, s_m1,
                           preferred_element_type=f32).astype(bf16)
            f_p1 = jnp.dot(feat, s_p1,
                           preferred_element_type=f32).astype(bf16)
            acc = jnp.dot(g_ref[0], f_m1, preferred_element_type=f32)
            acc = acc + jnp.dot(g_ref[1], feat, preferred_element_type=f32)
            acc = acc + jnp.dot(g_ref[2], f_p1, preferred_element_type=f32)
            o_ref[hb * part + g] = jnp.maximum(acc + bc_ref[...], 0.0)


def kernel(img, w_proj, b_proj, w_conv, b_conv):
    B, C, H, W = img.shape
    Hf, Wf = H // _POOL, W // _POOL
    Hc, Wc = Hf * _POOL, Wf * _POOL

    if (H, W) != (Hc, Wc):
        img = img[:, :, :Hc, :Wc]
    img = img.astype(jnp.float32)

    xt = jnp.swapaxes(img, 2, 3).reshape(B, C * Wc, Hc)

    ph = ((jnp.arange(Hc)[:, None] // _POOL == jnp.arange(Hf)[None, :])
          .astype(jnp.float32) / _POOL).astype(jnp.bfloat16)

    wp = w_proj.astype(jnp.float32) / _POOL
    eye_w = jnp.eye(Wf, dtype=jnp.float32)
    M = (jnp.einsum('cd,wk->wdck', wp, eye_w)
         .reshape(Wf * _FEAT_C, C * Wf).astype(jnp.bfloat16))
    bp_col = jnp.tile(b_proj.astype(jnp.float32),
                      Wf).reshape(Wf * _FEAT_C, 1)

    wc = w_conv.astype(jnp.float32)
    dxw = jnp.arange(Wf)[None, :] - jnp.arange(Wf)[:, None] + 1
    valid = ((dxw >= 0) & (dxw <= 2)).astype(jnp.float32)
    T = wc[:, jnp.clip(dxw, 0, 2)] * valid[None, :, :, None, None]
    G = (jnp.transpose(T, (0, 4, 1, 2, 3))
         .reshape(3, _OUT_C * Wf, Wf * _FEAT_C).astype(jnp.bfloat16))
    bc_col = jnp.repeat(b_conv.astype(jnp.float32),
                        Wf).reshape(_OUT_C * Wf, 1)

    body = functools.partial(_body, C, Hf, Wf)
    assert B % _GB == 0

    out_t = pl.pallas_call(
        body,
        out_shape=jax.ShapeDtypeStruct((B, _OUT_C * Wf, Hf), jnp.float32),
        grid_spec=pltpu.PrefetchScalarGridSpec(
            num_scalar_prefetch=0,
            grid=(B // _GB,),
            in_specs=[
                pl.BlockSpec((_GB // _NS, C * Wc, Hc),
                             lambda b: (_NS * b, 0, 0)),
                pl.BlockSpec((_GB // _NS, C * Wc, Hc),
                             lambda b: (_NS * b + 1, 0, 0)),
                pl.BlockSpec((_GB // _NS, C * Wc, Hc),
                             lambda b: (_NS * b + 2, 0, 0)),
                pl.BlockSpec((_GB // _NS, C * Wc, Hc),
                             lambda b: (_NS * b + 3, 0, 0)),
                pl.BlockSpec(ph.shape, lambda b: (0, 0)),
                pl.BlockSpec(M.shape, lambda b: (0, 0)),
                pl.BlockSpec(bp_col.shape, lambda b: (0, 0)),
                pl.BlockSpec(G.shape, lambda b: (0, 0, 0)),
                pl.BlockSpec(bc_col.shape, lambda b: (0, 0)),
            ],
            out_specs=pl.BlockSpec((_GB, _OUT_C * Wf, Hf),
                                   lambda b: (b, 0, 0)),
        ),
        compiler_params=pltpu.CompilerParams(
            dimension_semantics=("parallel",)),
    )(xt, xt, xt, xt, ph, M, bp_col, G, bc_col)

    return jnp.swapaxes(out_t.reshape(B, _OUT_C, Wf, Hf), 2, 3)


# speedup vs baseline: 1.0516x; 1.0516x over previous
import functools

import jax
import jax.numpy as jnp
from jax import lax
from jax.experimental import pallas as pl
from jax.experimental.pallas import tpu as pltpu

_FEAT_C = 64
_OUT_C = 32
_POOL = 64
_GB = 8
_NS = 2


def _body(C, Hf, Wf, x0_ref, x1_ref, ph_ref, m_ref, bp_ref,
          g_ref, bc_ref, o_ref):
    f32 = jnp.float32
    bf16 = jnp.bfloat16
    Hc = x0_ref.shape[-1]
    part = _GB // _NS
    CWf = C * Wf

    hi = lax.broadcasted_iota(jnp.int32, (Hf, Hf), 0)
    ho = lax.broadcasted_iota(jnp.int32, (Hf, Hf), 1)
    s_m1 = (hi == ho - 1).astype(bf16)
    s_p1 = (hi == ho + 1).astype(bf16)

    for hb, x_ref in enumerate((x0_ref, x1_ref)):
        x = x_ref[...]
        y = x.reshape(part * C * Wf, _POOL, Hc).sum(axis=1)

        xp = jnp.dot(y.astype(bf16), ph_ref[...],
                     preferred_element_type=f32)

        for g in range(part):
            xp_g = xp[g * CWf:(g + 1) * CWf, :].astype(bf16)
            f_pre = jnp.dot(m_ref[...], xp_g,
                            preferred_element_type=f32)
            feat = jnp.maximum(f_pre + bp_ref[...], 0.0).astype(bf16)

            f_m1 = jnp.dot(feat, s_m1,
                           preferred_element_type=f32).astype(bf16)
            f_p1 = jnp.dot(feat, s_p1,
                           preferred_element_type=f32).astype(bf16)
            acc = jnp.dot(g_ref[0], f_m1, preferred_element_type=f32)
            acc = acc + jnp.dot(g_ref[1], feat, preferred_element_type=f32)
            acc = acc + jnp.dot(g_ref[2], f_p1, preferred_element_type=f32)
            o_ref[hb * part + g] = jnp.maximum(acc + bc_ref[...], 0.0)


def kernel(img, w_proj, b_proj, w_conv, b_conv):
    B, C, H, W = img.shape
    Hf, Wf = H // _POOL, W // _POOL
    Hc, Wc = Hf * _POOL, Wf * _POOL

    if (H, W) != (Hc, Wc):
        img = img[:, :, :Hc, :Wc]
    img = img.astype(jnp.float32)

    xt = jnp.swapaxes(img, 2, 3).reshape(B, C * Wc, Hc)

    ph = ((jnp.arange(Hc)[:, None] // _POOL == jnp.arange(Hf)[None, :])
          .astype(jnp.float32) / _POOL).astype(jnp.bfloat16)

    wp = w_proj.astype(jnp.float32) / _POOL
    eye_w = jnp.eye(Wf, dtype=jnp.float32)
    M = (jnp.einsum('cd,wk->wdck', wp, eye_w)
         .reshape(Wf * _FEAT_C, C * Wf).astype(jnp.bfloat16))
    bp_col = jnp.tile(b_proj.astype(jnp.float32),
                      Wf).reshape(Wf * _FEAT_C, 1)

    wc = w_conv.astype(jnp.float32)
    dxw = jnp.arange(Wf)[None, :] - jnp.arange(Wf)[:, None] + 1
    valid = ((dxw >= 0) & (dxw <= 2)).astype(jnp.float32)
    T = wc[:, jnp.clip(dxw, 0, 2)] * valid[None, :, :, None, None]
    G = (jnp.transpose(T, (0, 4, 1, 2, 3))
         .reshape(3, _OUT_C * Wf, Wf * _FEAT_C).astype(jnp.bfloat16))
    bc_col = jnp.repeat(b_conv.astype(jnp.float32),
                        Wf).reshape(_OUT_C * Wf, 1)

    body = functools.partial(_body, C, Hf, Wf)
    assert B % _GB == 0

    out_t = pl.pallas_call(
        body,
        out_shape=jax.ShapeDtypeStruct((B, _OUT_C * Wf, Hf), jnp.float32),
        grid_spec=pltpu.PrefetchScalarGridSpec(
            num_scalar_prefetch=0,
            grid=(B // _GB,),
            in_specs=[
                pl.BlockSpec((_GB // _NS, C * Wc, Hc),
                             lambda b: (_NS * b, 0, 0)),
                pl.BlockSpec((_GB // _NS, C * Wc, Hc),
                             lambda b: (_NS * b + 1, 0, 0)),
                pl.BlockSpec(ph.shape, lambda b: (0, 0)),
                pl.BlockSpec(M.shape, lambda b: (0, 0)),
                pl.BlockSpec(bp_col.shape, lambda b: (0, 0)),
                pl.BlockSpec(G.shape, lambda b: (0, 0, 0)),
                pl.BlockSpec(bc_col.shape, lambda b: (0, 0)),
            ],
            out_specs=pl.BlockSpec((_GB, _OUT_C * Wf, Hf),
                                   lambda b: (b, 0, 0)),
        ),
        compiler_params=pltpu.CompilerParams(
            dimension_semantics=("parallel",)),
    )(xt, xt, ph, M, bp_col, G, bc_col)

    return jnp.swapaxes(out_t.reshape(B, _OUT_C, Wf, Hf), 2, 3)
